# baseline (speedup 1.0000x reference)
import functools

import jax
import jax.numpy as jnp
from jax import lax
from jax.experimental import pallas as pl
from jax.experimental.pallas import tpu as pltpu
from jax.experimental.pallas import tpu_sc as plsc

N = 2048
D = 768
E = 8
K = 2
EPAD = 128
T = 128
NT = N * K // T + E
NR = NT * T
NC = 2
NS = 16
NW = NC * NS


def _meta_body(x_ref, wg_ref, bg_ref, pos1_ref, pos2_ref, w1_ref, w2_ref,
               te_ref):
    idxv = lax.broadcasted_iota(jnp.int32, (N, EPAD), 1)
    logits = jnp.dot(x_ref[...], wg_ref[...],
                     preferred_element_type=jnp.float32) + bg_ref[...]
    m = jnp.max(logits, axis=1, keepdims=True)
    p = jnp.exp(logits - m)
    w = p / jnp.sum(p, axis=1, keepdims=True)
    m1 = jnp.max(w, axis=1, keepdims=True)
    i1 = jnp.min(jnp.where(w == m1, idxv, EPAD), axis=1, keepdims=True)
    wc = jnp.where(idxv == i1, -1.0, w)
    m2 = jnp.max(wc, axis=1, keepdims=True)
    i2 = jnp.min(jnp.where(wc == m2, idxv, EPAD), axis=1, keepdims=True)
    s = m1 + m2 + 1e-10
    w1_ref[...] = m1 / s
    w2_ref[...] = m2 / s

    oh1 = jnp.where(idxv == i1, 1.0, 0.0)
    oh2 = jnp.where(idxv == i2, 1.0, 0.0)

    rsub = lax.broadcasted_iota(jnp.int32, (T, T), 0)
    rlan = lax.broadcasted_iota(jnp.int32, (T, T), 1)
    l_strict = jnp.where(rlan < rsub, 1.0, 0.0)
    l_lane = jnp.where(rsub < rlan, 1.0, 0.0)

    def _rank(oh):
        off = jnp.zeros((1, EPAD), jnp.float32)
        blocks = []
        for b in range(N // T):
            blk = oh[b * T:(b + 1) * T, :]
            within = jnp.dot(l_strict, blk, preferred_element_type=jnp.float32)
            blocks.append(within + off)
            off = off + jnp.sum(blk, axis=0, keepdims=True)
        return off, jnp.concatenate(blocks, axis=0)

    cnt1, rank1 = _rank(oh1)
    cnt2, rank2 = _rank(oh2)
    rank2 = rank2 + cnt1
    cnt = cnt1 + cnt2

    tiles = jnp.floor((cnt + (T - 1.0)) * (1.0 / T))
    tile_start = jnp.dot(tiles, l_lane, preferred_element_type=jnp.float32)
    row_start = tile_start * T

    pos1 = jnp.sum(oh1 * (row_start + rank1), axis=1, keepdims=True)
    pos2 = jnp.sum(oh2 * (row_start + rank2), axis=1, keepdims=True)
    pos1_ref[...] = pos1.astype(jnp.int32)
    pos2_ref[...] = pos2.astype(jnp.int32)

    ts_t = jnp.transpose(jnp.broadcast_to(tile_start, (EPAD, EPAD)))[:, :1]
    jlane = lax.broadcasted_iota(jnp.int32, (EPAD, EPAD), 1).astype(jnp.float32)
    cmp = jnp.where(ts_t <= jlane, 1.0, 0.0)
    te = jnp.dot(jnp.ones((1, EPAD), jnp.float32), cmp,
                 preferred_element_type=jnp.float32) - 1.0
    te_ref[...] = jnp.clip(te, 0.0, E - 1.0).astype(jnp.int32)


def _route_meta(x, wg_pad, bg_pad):
    return pl.pallas_call(
        _meta_body,
        grid=(1,),
        in_specs=[
            pl.BlockSpec((N, D), lambda i: (0, 0)),
            pl.BlockSpec((D, EPAD), lambda i: (0, 0)),
            pl.BlockSpec((1, EPAD), lambda i: (0, 0)),
        ],
        out_specs=(
            pl.BlockSpec((N, 1), lambda i: (0, 0)),
            pl.BlockSpec((N, 1), lambda i: (0, 0)),
            pl.BlockSpec((N, 1), lambda i: (0, 0)),
            pl.BlockSpec((N, 1), lambda i: (0, 0)),
            pl.BlockSpec((1, EPAD), lambda i: (0, 0)),
        ),
        out_shape=(
            jax.ShapeDtypeStruct((N, 1), jnp.int32),
            jax.ShapeDtypeStruct((N, 1), jnp.int32),
            jax.ShapeDtypeStruct((N, 1), jnp.float32),
            jax.ShapeDtypeStruct((N, 1), jnp.float32),
            jax.ShapeDtypeStruct((1, EPAD), jnp.int32),
        ),
    )(x, wg_pad, bg_pad)


_PW = N // NW


def _scatter_meta(pos1, pos2, w1, w2):
    mesh = plsc.VectorSubcoreMesh(core_axis_name="c", subcore_axis_name="s")

    @functools.partial(
        pl.kernel, mesh=mesh,
        out_type=(
            jax.ShapeDtypeStruct((NR,), jnp.int32),
            jax.ShapeDtypeStruct((NR,), jnp.float32),
        ),
        scratch_types=[
            pltpu.VMEM((_PW,), jnp.int32),
            pltpu.VMEM((_PW,), jnp.int32),
            pltpu.VMEM((_PW,), jnp.int32),
            pltpu.VMEM((_PW,), jnp.float32),
            pltpu.VMEM((_PW,), jnp.float32),
            pltpu.SemaphoreType.DMA,
        ],
    )
    def k(pos1_hbm, pos2_hbm, w1_hbm, w2_hbm, stok_hbm, wsrt_hbm,
          tok_v, idx1_v, idx2_v, w1_v, w2_v, sem):
        wid = lax.axis_index("s") * NC + lax.axis_index("c")
        base = wid * _PW
        for j in range(_PW // 16):
            tok_v[pl.ds(j * 16, 16)] = base + j * 16 + lax.iota(jnp.int32, 16)
        pltpu.sync_copy(pos1_hbm.at[pl.ds(base, _PW)], idx1_v)
        pltpu.sync_copy(pos2_hbm.at[pl.ds(base, _PW)], idx2_v)
        pltpu.sync_copy(w1_hbm.at[pl.ds(base, _PW)], w1_v)
        pltpu.sync_copy(w2_hbm.at[pl.ds(base, _PW)], w2_v)
        c1 = pltpu.async_copy(tok_v, stok_hbm.at[idx1_v], sem)
        c2 = pltpu.async_copy(tok_v, stok_hbm.at[idx2_v], sem)
        c3 = pltpu.async_copy(w1_v, wsrt_hbm.at[idx1_v], sem)
        c4 = pltpu.async_copy(w2_v, wsrt_hbm.at[idx2_v], sem)
        c1.wait()
        c2.wait()
        c3.wait()
        c4.wait()

    return k(pos1, pos2, w1, w2)


_RW = NR // NW
_GC = 16


def _gather_x(x, stok):
    mesh = plsc.VectorSubcoreMesh(core_axis_name="c", subcore_axis_name="s")

    @functools.partial(
        pl.kernel, mesh=mesh,
        out_type=jax.ShapeDtypeStruct((NR, D), jnp.float32),
        scratch_types=[
            pltpu.VMEM((_RW,), jnp.int32),
            pltpu.VMEM((_RW, D), jnp.float32),
            pltpu.SemaphoreType.DMA,
        ],
    )
    def k(x_hbm, stok_hbm, xg_hbm, idx_v, rows_v, sem):
        wid = lax.axis_index("s") * NC + lax.axis_index("c")
        base = wid * _RW
        pltpu.sync_copy(stok_hbm.at[pl.ds(base, _RW)], idx_v)
        for j in range(_RW // 16):
            sl = pl.ds(j * 16, 16)
            idx_v[sl] = jnp.clip(idx_v[sl], 0, N - 1)
        copies = [
            pltpu.async_copy(x_hbm.at[idx_v.at[pl.ds(c * _GC, _GC)]],
                             rows_v.at[pl.ds(c * _GC, _GC)], sem)
            for c in range(_RW // _GC)
        ]
        for c in copies:
            c.wait()
        pltpu.sync_copy(rows_v, xg_hbm.at[pl.ds(base, _RW)])

    return k(x, stok)


def _gmm_body(te_ref, xg_ref, we_ref, be_ref, wrow_ref, y_ref):
    xb = xg_ref[...].astype(jnp.bfloat16)
    y = jnp.dot(xb, we_ref[0], preferred_element_type=jnp.float32)
    y = y + be_ref[0]
    wcol = jnp.transpose(wrow_ref[0])
    y_ref[...] = y * wcol


def _gmm(xg, we_bf, be, wsrt, te):
    grid_spec = pltpu.PrefetchScalarGridSpec(
        num_scalar_prefetch=1,
        grid=(NT,),
        in_specs=[
            pl.BlockSpec((T, D), lambda i, te: (i, 0)),
            pl.BlockSpec((1, D, D), lambda i, te: (te[i], 0, 0)),
            pl.BlockSpec((1, 1, D), lambda i, te: (te[i], 0, 0)),
            pl.BlockSpec((1, 1, T), lambda i, te: (i, 0, 0)),
        ],
        out_specs=pl.BlockSpec((T, D), lambda i, te: (i, 0)),
    )
    return pl.pallas_call(
        _gmm_body,
        grid_spec=grid_spec,
        out_shape=jax.ShapeDtypeStruct((NR, D), jnp.float32),
    )(te, xg, we_bf, be.reshape(E, 1, D), wsrt.reshape(NT, 1, T))


_TW = N // NW
_TH = _TW // 2


def _combine(y, pos1, pos2):
    mesh = plsc.VectorSubcoreMesh(core_axis_name="c", subcore_axis_name="s")

    @functools.partial(
        pl.kernel, mesh=mesh,
        out_type=jax.ShapeDtypeStruct((N, D), jnp.float32),
        scratch_types=[
            pltpu.VMEM((_TW,), jnp.int32),
            pltpu.VMEM((_TW,), jnp.int32),
            pltpu.VMEM((_TH, D), jnp.float32),
            pltpu.VMEM((_TH, D), jnp.float32),
            pltpu.SemaphoreType.DMA,
        ],
    )
    def k(y_hbm, pos1_hbm, pos2_hbm, out_hbm, p1_v, p2_v, ra_v, rb_v, sem):
        wid = lax.axis_index("s") * NC + lax.axis_index("c")
        base = wid * _TW
        pltpu.sync_copy(pos1_hbm.at[pl.ds(base, _TW)], p1_v)
        pltpu.sync_copy(pos2_hbm.at[pl.ds(base, _TW)], p2_v)
        for h in range(2):
            copies = [
                pltpu.async_copy(
                    y_hbm.at[p1_v.at[pl.ds(h * _TH + c * _GC, _GC)]],
                    ra_v.at[pl.ds(c * _GC, _GC)], sem)
                for c in range(_TH // _GC)
            ] + [
                pltpu.async_copy(
                    y_hbm.at[p2_v.at[pl.ds(h * _TH + c * _GC, _GC)]],
                    rb_v.at[pl.ds(c * _GC, _GC)], sem)
                for c in range(_TH // _GC)
            ]
            for c in copies:
                c.wait()

            def row(r, _):
                for c in range(D // 16):
                    sl = pl.ds(c * 16, 16)
                    ra_v[r, sl] = ra_v[r, sl] + rb_v[r, sl]
                return 0
            lax.fori_loop(0, _TH, row, 0)
            pltpu.sync_copy(ra_v, out_hbm.at[pl.ds(base + h * _TH, _TH)])

    return k(y, pos1, pos2)


@jax.jit
def kernel(x, Wg, bg, We, be):
    wg_pad = jnp.zeros((D, EPAD), jnp.float32).at[:, :E].set(Wg)
    bg_pad = jnp.full((1, EPAD), -1e30, jnp.float32).at[0, :E].set(bg)
    we_bf = We.astype(jnp.bfloat16)

    pos1, pos2, w1, w2, te = _route_meta(x, wg_pad, bg_pad)
    pos1 = pos1.reshape(N)
    pos2 = pos2.reshape(N)
    stok, wsrt = _scatter_meta(pos1, pos2, w1.reshape(N), w2.reshape(N))
    xg = _gather_x(x, stok)
    y = _gmm(xg, we_bf, be, wsrt, te.reshape(EPAD)[:NT])
    out = _combine(y, pos1, pos2)
    return out


# speedup vs baseline: 3.9197x; 3.9197x over previous
import jax
import jax.numpy as jnp
from jax import lax
from jax.experimental import pallas as pl
from jax.experimental.pallas import tpu as pltpu

N = 2048
D = 768
E = 8
K = 2
EPAD = 128
NCHUNK = 2
NB = N // NCHUNK


def _moe_body(x_ref, wg_ref, bg_ref, ws_ref, out_ref):
    idxv = lax.broadcasted_iota(jnp.int32, (NB, EPAD), 1)
    x = x_ref[...]
    logits = jnp.dot(x, wg_ref[...],
                     preferred_element_type=jnp.float32) + bg_ref[...]
    m = jnp.max(logits, axis=1, keepdims=True)
    p = jnp.exp(logits - m)
    w = p / jnp.sum(p, axis=1, keepdims=True)
    m1 = jnp.max(w, axis=1, keepdims=True)
    i1 = jnp.min(jnp.where(w == m1, idxv, EPAD), axis=1, keepdims=True)
    wc = jnp.where(idxv == i1, -1.0, w)
    m2 = jnp.max(wc, axis=1, keepdims=True)
    i2 = jnp.min(jnp.where(wc == m2, idxv, EPAD), axis=1, keepdims=True)
    sel = (idxv == i1) | (idxv == i2)
    wsel = jnp.where(sel, w, 0.0)
    wfull = wsel / (jnp.sum(wsel, axis=1, keepdims=True) + 1e-10)

    xw = jnp.concatenate(
        [(lax.slice(wfull, (0, e), (NB, e + 1)) * x).astype(jnp.bfloat16)
         for e in range(E)] + [wfull.astype(jnp.bfloat16)],
        axis=1)
    out_ref[...] = jnp.dot(xw, ws_ref[...],
                           preferred_element_type=jnp.float32)


@jax.jit
def kernel(x, Wg, bg, We, be):
    wg_pad = jnp.zeros((D, EPAD), jnp.float32).at[:, :E].set(Wg)
    bg_pad = jnp.full((1, EPAD), -1e30, jnp.float32).at[0, :E].set(bg)
    be_pad = jnp.zeros((EPAD, D), jnp.bfloat16).at[:E, :].set(
        be.astype(jnp.bfloat16))
    ws = jnp.concatenate(
        [We.astype(jnp.bfloat16).reshape(E * D, D), be_pad], axis=0)

    out = pl.pallas_call(
        _moe_body,
        grid=(NCHUNK,),
        in_specs=[
            pl.BlockSpec((NB, D), lambda i: (i, 0)),
            pl.BlockSpec((D, EPAD), lambda i: (0, 0)),
            pl.BlockSpec((1, EPAD), lambda i: (0, 0)),
            pl.BlockSpec((E * D + EPAD, D), lambda i: (0, 0)),
        ],
        out_specs=pl.BlockSpec((NB, D), lambda i: (i, 0)),
        out_shape=jax.ShapeDtypeStruct((N, D), jnp.float32),
    )(x, wg_pad, bg_pad, ws)
    return out


# speedup vs baseline: 4.1097x; 1.0485x over previous
import jax
import jax.numpy as jnp
from jax import lax
from jax.experimental import pallas as pl
from jax.experimental.pallas import tpu as pltpu

N = 2048
D = 768
E = 8
K = 2
EPAD = 128
NCHUNK = 4
NB = N // NCHUNK


def _moe_body(x_ref, wg_ref, bg_ref, ws_ref, be_ref, out_ref):
    idxv = lax.broadcasted_iota(jnp.int32, (NB, EPAD), 1)
    x = x_ref[...]
    logits = jnp.dot(x, wg_ref[...],
                     preferred_element_type=jnp.float32) + bg_ref[...]
    m = jnp.max(logits, axis=1, keepdims=True)
    p = jnp.exp(logits - m)
    w = p / jnp.sum(p, axis=1, keepdims=True)
    m1 = jnp.max(w, axis=1, keepdims=True)
    i1 = jnp.min(jnp.where(w == m1, idxv, EPAD), axis=1, keepdims=True)
    wc = jnp.where(idxv == i1, -1.0, w)
    m2 = jnp.max(wc, axis=1, keepdims=True)
    i2 = jnp.min(jnp.where(wc == m2, idxv, EPAD), axis=1, keepdims=True)
    sel = (idxv == i1) | (idxv == i2)
    wsel = jnp.where(sel, w, 0.0)
    wfull = wsel / (jnp.sum(wsel, axis=1, keepdims=True) + 1e-10)

    xw = jnp.concatenate(
        [(lax.slice(wfull, (0, e), (NB, e + 1)) * x).astype(jnp.bfloat16)
         for e in range(E)], axis=1)
    out_ref[...] = (
        jnp.dot(xw, ws_ref[...], preferred_element_type=jnp.float32)
        + jnp.dot(wfull, be_ref[...], preferred_element_type=jnp.float32))


@jax.jit
def kernel(x, Wg, bg, We, be):
    wg_pad = jnp.zeros((D, EPAD), jnp.float32).at[:, :E].set(Wg)
    bg_pad = jnp.full((1, EPAD), -1e30, jnp.float32).at[0, :E].set(bg)
    be_pad = jnp.zeros((EPAD, D), jnp.float32).at[:E, :].set(be)
    ws = We.astype(jnp.bfloat16).reshape(E * D, D)

    out = pl.pallas_call(
        _moe_body,
        grid=(NCHUNK,),
        in_specs=[
            pl.BlockSpec((NB, D), lambda i: (i, 0)),
            pl.BlockSpec((D, EPAD), lambda i: (0, 0)),
            pl.BlockSpec((1, EPAD), lambda i: (0, 0)),
            pl.BlockSpec((E * D, D), lambda i: (0, 0)),
            pl.BlockSpec((EPAD, D), lambda i: (0, 0)),
        ],
        out_specs=pl.BlockSpec((NB, D), lambda i: (i, 0)),
        out_shape=jax.ShapeDtypeStruct((N, D), jnp.float32),
    )(x, wg_pad, bg_pad, ws, be_pad)
    return out


# speedup vs baseline: 4.1919x; 1.0200x over previous
import jax
import jax.numpy as jnp
from jax import lax
from jax.experimental import pallas as pl
from jax.experimental.pallas import tpu as pltpu

N = 2048
D = 768
E = 8
K = 2
EPAD = 128
NCHUNK = 2
NB = N // NCHUNK


def _moe_body(x_ref, wg_ref, bg_ref, ws_ref, be_ref, out_ref):
    idxv = lax.broadcasted_iota(jnp.int32, (NB, EPAD), 1)
    x = x_ref[...]
    logits = jnp.dot(x, wg_ref[...],
                     preferred_element_type=jnp.float32) + bg_ref[...]
    m = jnp.max(logits, axis=1, keepdims=True)
    p = jnp.exp(logits - m)
    w = p / jnp.sum(p, axis=1, keepdims=True)
    m1 = jnp.max(w, axis=1, keepdims=True)
    i1 = jnp.min(jnp.where(w == m1, idxv, EPAD), axis=1, keepdims=True)
    wc = jnp.where(idxv == i1, -1.0, w)
    m2 = jnp.max(wc, axis=1, keepdims=True)
    i2 = jnp.min(jnp.where(wc == m2, idxv, EPAD), axis=1, keepdims=True)
    sel = (idxv == i1) | (idxv == i2)
    wsel = jnp.where(sel, w, 0.0)
    wfull = wsel / (jnp.sum(wsel, axis=1, keepdims=True) + 1e-10)

    xw = jnp.concatenate(
        [(lax.slice(wfull, (0, e), (NB, e + 1)) * x).astype(jnp.bfloat16)
         for e in range(E)], axis=1)
    out_ref[...] = (
        jnp.dot(xw, ws_ref[...], preferred_element_type=jnp.float32)
        + jnp.dot(wfull, be_ref[...], preferred_element_type=jnp.float32))


@jax.jit
def kernel(x, Wg, bg, We, be):
    wg_pad = jnp.zeros((D, EPAD), jnp.float32).at[:, :E].set(Wg)
    bg_pad = jnp.full((1, EPAD), -1e30, jnp.float32).at[0, :E].set(bg)
    be_pad = jnp.zeros((EPAD, D), jnp.float32).at[:E, :].set(be)
    ws = We.astype(jnp.bfloat16).reshape(E * D, D)

    out = pl.pallas_call(
        _moe_body,
        grid=(NCHUNK,),
        in_specs=[
            pl.BlockSpec((NB, D), lambda i: (i, 0)),
            pl.BlockSpec((D, EPAD), lambda i: (0, 0)),
            pl.BlockSpec((1, EPAD), lambda i: (0, 0)),
            pl.BlockSpec((E * D, D), lambda i: (0, 0)),
            pl.BlockSpec((EPAD, D), lambda i: (0, 0)),
        ],
        out_specs=pl.BlockSpec((NB, D), lambda i: (i, 0)),
        out_shape=jax.ShapeDtypeStruct((N, D), jnp.float32),
    )(x, wg_pad, bg_pad, ws, be_pad)
    return out
